# baseline (speedup 1.0000x reference)
import functools

import jax
import jax.numpy as jnp
from jax import lax
from jax.experimental import pallas as pl
from jax.experimental.pallas import tpu as pltpu
from jax.experimental.pallas import tpu_sc as plsc

Nv_ = 100000
Nh_ = 16
BATCH_ = 4096
XMIN_ = -10.0
XMAX_ = 10.0
DX_ = (XMAX_ - XMIN_) / (Nv_ - 1)

_NC = 2
_NS = 16
_NW = _NC * _NS
_BPW = BATCH_ // _NW
_L = 16


def _rbm_body(x_hbm, b_hbm, c_hbm, wt_hbm, out_hbm,
              x_v, idx2_v, t_v, b_v, c_v, ch_v, out_v, sem_w, sem_b):
    wid = lax.axis_index("s") * _NC + lax.axis_index("c")
    base = wid * _BPW

    pltpu.sync_copy(x_hbm.at[pl.ds(base, _BPW)], x_v)
    pltpu.sync_copy(c_hbm, c_v)

    def idx_body(k, carry):
        off = pl.multiple_of(k * _L, _L)
        xv = x_v[pl.ds(off, _L)]
        ii = ((xv - XMIN_) / DX_).astype(jnp.int32)
        ii = jnp.minimum(jnp.maximum(ii, 0), Nv_ - 1)
        for h in range(Nh_):
            idx2_v[h, pl.ds(off, _L)] = ii + h * Nv_
        return carry

    lax.fori_loop(0, _BPW // _L, idx_body, 0, unroll=False)

    cp_b = pltpu.async_copy(b_hbm.at[idx2_v.at[0]], b_v, sem_b)
    cps = [pltpu.async_copy(wt_hbm.at[idx2_v.at[h]], t_v.at[h], sem_w)
           for h in range(Nh_)]

    lane16 = lax.iota(jnp.int32, _L) * _L
    cv = c_v[...]
    for j in range(_L):
        plsc.store_scatter(ch_v, [lane16 + j], cv)

    cp_b.wait()

    def eb_body(k, carry):
        off = pl.multiple_of(k * _L, _L)
        out_v[pl.ds(off, _L)] = jnp.exp(b_v[pl.ds(off, _L)])
        return carry

    lax.fori_loop(0, _BPW // _L, eb_body, 0, unroll=False)

    for h in range(Nh_):
        cps[h].wait()
        ch = ch_v[pl.ds(h * _L, _L)]

        def plane_body(k, carry, h=h, ch=ch):
            off = pl.multiple_of(k * _L, _L)
            out_v[pl.ds(off, _L)] = out_v[pl.ds(off, _L)] * (
                1.0 + jnp.exp(ch + t_v[h, pl.ds(off, _L)]))
            return carry

        lax.fori_loop(0, _BPW // _L, plane_body, 0, unroll=False)

    pltpu.sync_copy(out_v, out_hbm.at[pl.ds(base, _BPW)])


_SCRATCH = [
    pltpu.VMEM((_BPW,), jnp.float32),
    pltpu.VMEM((Nh_, _BPW), jnp.int32),
    pltpu.VMEM((Nh_, _BPW), jnp.float32),
    pltpu.VMEM((_BPW,), jnp.float32),
    pltpu.VMEM((Nh_,), jnp.float32),
    pltpu.VMEM((Nh_ * _L,), jnp.float32),
    pltpu.VMEM((_BPW,), jnp.float32),
    pltpu.SemaphoreType.DMA,
    pltpu.SemaphoreType.DMA,
]


def _prep(x, b, c, w):
    return x, b, c, w.T.reshape(-1)


_rbm_sc = functools.partial(
    pl.kernel,
    out_type=jax.ShapeDtypeStruct((BATCH_,), jnp.float32),
    mesh=plsc.VectorSubcoreMesh(core_axis_name="c", subcore_axis_name="s"),
    compiler_params=pltpu.CompilerParams(needs_layout_passes=False,
                                         use_tc_tiling_on_sc=False),
    scratch_types=_SCRATCH,
)(_rbm_body)


def kernel(x, b, c, w):
    return _rbm_sc(*_prep(x, b, c, w))


# speedup vs baseline: 1.0179x; 1.0179x over previous
import functools

import jax
import jax.numpy as jnp
from jax import lax
from jax.experimental import pallas as pl
from jax.experimental.pallas import tpu as pltpu
from jax.experimental.pallas import tpu_sc as plsc

Nv_ = 100000
Nh_ = 16
BATCH_ = 4096
XMIN_ = -10.0
XMAX_ = 10.0
DX_ = (XMAX_ - XMIN_) / (Nv_ - 1)

_NC = 2
_NS = 16
_NW = _NC * _NS
_BPW = BATCH_ // _NW
_L = 16


def _rbm_body(x_hbm, b_hbm, c_hbm, wt_hbm, out_hbm,
              x_v, idx2_v, t_v, b_v, c_v, ch_v, out_v, sem_w, sem_b):
    wid = lax.axis_index("s") * _NC + lax.axis_index("c")
    base = wid * _BPW

    pltpu.sync_copy(x_hbm.at[pl.ds(base, _BPW)], x_v)
    pltpu.sync_copy(c_hbm, c_v)

    def idx_body(k, carry):
        off = pl.multiple_of(k * _L, _L)
        xv = x_v[pl.ds(off, _L)]
        ii = ((xv - XMIN_) / DX_).astype(jnp.int32)
        ii = jnp.minimum(jnp.maximum(ii, 0), Nv_ - 1)
        for h in range(Nh_):
            idx2_v[h, pl.ds(off, _L)] = ii + h * Nv_
        return carry

    lax.fori_loop(0, _BPW // _L, idx_body, 0, unroll=False)

    cp_b = pltpu.async_copy(b_hbm.at[idx2_v.at[0]], b_v, sem_b)
    cps = [pltpu.async_copy(wt_hbm.at[idx2_v.at[h]], t_v.at[h], sem_w)
           for h in range(Nh_)]

    lane16 = lax.iota(jnp.int32, _L) * _L
    cv = c_v[...]
    for j in range(_L):
        plsc.store_scatter(ch_v, [lane16 + j], cv)

    cp_b.wait()
    for cp in cps:
        cp.wait()

    chs = [ch_v[pl.ds(h * _L, _L)] for h in range(Nh_)]

    def chunk_body(k, carry):
        off = pl.multiple_of(k * _L, _L)
        acc = jnp.exp(b_v[pl.ds(off, _L)])
        for h in range(Nh_):
            acc = acc * (1.0 + jnp.exp(chs[h] + t_v[h, pl.ds(off, _L)]))
        out_v[pl.ds(off, _L)] = acc
        return carry

    lax.fori_loop(0, _BPW // _L, chunk_body, 0, unroll=False)

    pltpu.sync_copy(out_v, out_hbm.at[pl.ds(base, _BPW)])


_SCRATCH = [
    pltpu.VMEM((_BPW,), jnp.float32),
    pltpu.VMEM((Nh_, _BPW), jnp.int32),
    pltpu.VMEM((Nh_, _BPW), jnp.float32),
    pltpu.VMEM((_BPW,), jnp.float32),
    pltpu.VMEM((Nh_,), jnp.float32),
    pltpu.VMEM((Nh_ * _L,), jnp.float32),
    pltpu.VMEM((_BPW,), jnp.float32),
    pltpu.SemaphoreType.DMA,
    pltpu.SemaphoreType.DMA,
]


def _prep(x, b, c, w):
    return x, b, c, w.T.reshape(-1)


_rbm_sc = functools.partial(
    pl.kernel,
    out_type=jax.ShapeDtypeStruct((BATCH_,), jnp.float32),
    mesh=plsc.VectorSubcoreMesh(core_axis_name="c", subcore_axis_name="s"),
    compiler_params=pltpu.CompilerParams(needs_layout_passes=False,
                                         use_tc_tiling_on_sc=False),
    scratch_types=_SCRATCH,
)(_rbm_body)


def kernel(x, b, c, w):
    return _rbm_sc(*_prep(x, b, c, w))
